# baseline (speedup 1.0000x reference)
import functools

import jax
import jax.numpy as jnp
from jax import lax
from jax.experimental import pallas as pl
from jax.experimental.pallas import tpu as pltpu
from jax.experimental.pallas import tpu_sc as plsc

N = 10000
E = 320000
D = 128
H = 16
NC = 2
NS = 16
L = 16
NW = NC * NS
EPW = E // NW
NBUF = 8
RPW = 80
EPW_PAD = RPW * 128
NPAD = RPW * 128
ZROWS = NPAD // NS
OUT = RPW // NBUF
NROW16 = NPAD // L
SROW = ZROWS // L
NCHUNK = NROW16 // 128

_mesh = plsc.VectorSubcoreMesh(core_axis_name="c", subcore_axis_name="s")
_sc_params = pltpu.CompilerParams(needs_layout_passes=False,
                                  use_tc_tiling_on_sc=False)


def _zero_1d(ref, nvecs):
    zero = jnp.zeros((L,), jnp.float32)

    def body(i, _):
        ref[pl.ds(i * L, L)] = zero
        return 0

    lax.fori_loop(0, nvecs, body, 0)


def _rsqrt16(d):
    i = plsc.bitcast(d, jnp.int32)
    i = jnp.int32(0x5F3759DF) - (i >> 1)
    x = plsc.bitcast(i, jnp.float32)
    for _ in range(3):
        x = x * (1.5 - 0.5 * d * x * x)
    return x


def _tc_xw_body(x_ref, w1_ref, xw_ref):
    xw_ref[:N] = jnp.dot(x_ref[...], w1_ref[...],
                         preferred_element_type=jnp.float32)
    xw_ref[N:] = jnp.zeros((NPAD - N, H), jnp.float32)


_tc_xw = pl.pallas_call(
    _tc_xw_body,
    out_shape=jax.ShapeDtypeStruct((NPAD, H), jnp.float32),
)


@functools.partial(
    pl.kernel,
    out_type=(
        jax.ShapeDtypeStruct((NW, NPAD), jnp.float32),
        jax.ShapeDtypeStruct((NC, NPAD, H), jnp.float32),
        jax.ShapeDtypeStruct((NROW16, L), jnp.float32),
    ),
    mesh=_mesh,
    compiler_params=_sc_params,
    scratch_types=[
        pltpu.VMEM((RPW, 128), jnp.int32),
        pltpu.VMEM((RPW, 128), jnp.int32),
        pltpu.VMEM((NROW16, L), jnp.float32),
        pltpu.VMEM((NPAD,), jnp.float32),
        pltpu.VMEM((NBUF, 128, H), jnp.float32),
        pltpu.VMEM((ZROWS, H), jnp.float32),
        pltpu.VMEM((SROW, L), jnp.float32),
        pltpu.VMEM((2 * RPW, 128), jnp.int32),
        pltpu.VMEM((NROW16, L), jnp.float32),
        pltpu.VMEM((NCHUNK, 128), jnp.int32),
        pltpu.VMEM_SHARED((NPAD, H), jnp.float32),
        pltpu.VMEM_SHARED((NPAD, H), jnp.float32),
        pltpu.VMEM_SHARED((NROW16, L), jnp.float32),
        pltpu.VMEM_SHARED((NROW16, L), jnp.float32),
        pltpu.SemaphoreType.DMA((NBUF,)),
        pltpu.SemaphoreType.DMA((NBUF,)),
    ],
)
def _sc_edge(src3_hbm, dst3_hbm, xw_hbm,
             t1_out, z_out, dinv_out,
             src_r, dst_r, dinv_v, t1_v, rows_v, sl16_v, ds2_v,
             dd_v, dg2_v, idxr_v, z_acc, y_sh, dinv_sh, deg_sh, gsem, ssem):
    cid = lax.axis_index("c")
    sid = lax.axis_index("s")
    wid = sid * NC + cid
    base = sid * ZROWS

    zrow = jnp.zeros((L,), jnp.float32)

    def zbody(i, _):
        sl16_v[i] = zrow
        return 0

    lax.fori_loop(0, ZROWS, zbody, 0)
    pltpu.sync_copy(sl16_v, z_acc.at[pl.ds(base, ZROWS)])

    def z2body(i, _):
        ds2_v[i] = zrow
        return 0

    lax.fori_loop(0, SROW, z2body, 0)
    pltpu.sync_copy(ds2_v, deg_sh.at[pl.ds(sid * SROW, SROW)])

    pltpu.sync_copy(dst3_hbm.at[2 * sid], dd_v.at[pl.ds(0, RPW)])
    pltpu.sync_copy(dst3_hbm.at[2 * sid + 1], dd_v.at[pl.ds(RPW, RPW)])

    def zdg(i, _):
        dg2_v[i] = zrow
        return 0

    lax.fori_loop(0, NROW16, zdg, 0)
    ones = jnp.ones((L,), jnp.float32)
    iota = lax.iota(jnp.int32, L)
    for c in range(NCHUNK):
        for k in range(128 // L):
            idxr_v[c, pl.ds(k * L, L)] = iota + (c * 128 + k * L)

    def dbody(j, _):
        for k in range(128 // L):
            idx = dd_v[j, pl.ds(k * L, L)]
            plsc.addupdate_scatter(dg2_v, [idx >> 4, idx & 15], ones)
        return 0

    lax.fori_loop(0, 2 * RPW, dbody, 0)
    plsc.subcore_barrier()

    for c in range(NCHUNK):
        pltpu.sync_copy(dg2_v.at[pl.ds(c * 128, 128)],
                        deg_sh.at[idxr_v.at[c]], add=True)
    plsc.subcore_barrier()

    pltpu.sync_copy(deg_sh.at[pl.ds(sid * SROW, SROW)], ds2_v)
    for i in range(SROW):
        d = ds2_v[i] + 1.0
        r = _rsqrt16(d)
        mask = (iota + (base + i * L)) < N
        ds2_v[i] = jnp.where(mask, r, 0.0)
    pltpu.sync_copy(ds2_v, dinv_sh.at[pl.ds(sid * SROW, SROW)])

    @pl.when(cid == 0)
    def _():
        pltpu.sync_copy(ds2_v, dinv_out.at[pl.ds(sid * SROW, SROW)])

    pltpu.sync_copy(xw_hbm.at[pl.ds(base, ZROWS)], sl16_v)

    def ybody(i, _):
        dv = ds2_v[i]
        for k in range(L):
            r = i * L + k
            sl16_v[r] = sl16_v[r] * dv[k]
        return 0

    lax.fori_loop(0, SROW, ybody, 0)
    pltpu.sync_copy(sl16_v, y_sh.at[pl.ds(base, ZROWS)])
    plsc.subcore_barrier()

    pltpu.sync_copy(dinv_sh, dinv_v)
    pltpu.sync_copy(src3_hbm.at[wid], src_r)
    pltpu.sync_copy(dst3_hbm.at[wid], dst_r)
    _zero_1d(t1_v, NPAD // L)

    def _t1_row(j):
        for k in range(128 // L):
            d_idx = dst_r[j, pl.ds(k * L, L)]
            s_idx = src_r[j, pl.ds(k * L, L)]
            vals = plsc.load_gather(dinv_v, [d_idx >> 4, d_idx & 15])
            plsc.addupdate_scatter(t1_v, [s_idx], vals)

    for b in range(NBUF):
        pltpu.async_copy(y_sh.at[src_r.at[b]], rows_v.at[b], gsem.at[b])

    def pipe_body(o, _):
        for b in range(NBUF):
            j = o * NBUF + b
            _t1_row(j)
            pltpu.make_async_copy(
                y_sh.at[src_r.at[j]], rows_v.at[b], gsem.at[b]).wait()
            pltpu.async_copy(rows_v.at[b], z_acc.at[dst_r.at[j]],
                             ssem.at[b], add=True)
            pltpu.make_async_copy(
                rows_v.at[b], z_acc.at[dst_r.at[j]], ssem.at[b]).wait()
            pltpu.async_copy(y_sh.at[src_r.at[j + NBUF]], rows_v.at[b],
                             gsem.at[b])
        return 0

    lax.fori_loop(0, OUT - 1, pipe_body, 0)
    for b in range(NBUF):
        j = (OUT - 1) * NBUF + b
        _t1_row(j)
        pltpu.make_async_copy(
            y_sh.at[src_r.at[j]], rows_v.at[b], gsem.at[b]).wait()
        pltpu.async_copy(rows_v.at[b], z_acc.at[dst_r.at[j]],
                         ssem.at[b], add=True)
        pltpu.make_async_copy(
            rows_v.at[b], z_acc.at[dst_r.at[j]], ssem.at[b]).wait()

    pltpu.sync_copy(t1_v, t1_out.at[wid])
    plsc.subcore_barrier()
    pltpu.sync_copy(z_acc.at[pl.ds(base, ZROWS)], sl16_v)
    pltpu.sync_copy(sl16_v, z_out.at[cid].at[pl.ds(base, ZROWS)])


@functools.partial(
    pl.kernel,
    out_type=(
        jax.ShapeDtypeStruct((NW, NPAD), jnp.float32),
        jax.ShapeDtypeStruct((NROW16, L), jnp.float32),
    ),
    mesh=_mesh,
    compiler_params=_sc_params,
    scratch_types=[
        pltpu.VMEM((EPW_PAD,), jnp.int32),
        pltpu.VMEM((EPW_PAD,), jnp.int32),
        pltpu.VMEM((NROW16, L), jnp.float32),
        pltpu.VMEM((NPAD,), jnp.float32),
        pltpu.VMEM((NW, ZROWS), jnp.float32),
        pltpu.VMEM((ZROWS,), jnp.float32),
        pltpu.VMEM((SROW, L), jnp.float32),
        pltpu.VMEM((SROW, L), jnp.float32),
        pltpu.VMEM_SHARED((NROW16, L), jnp.float32),
    ],
)
def _sc_t2(src_hbm, dst_hbm, t1p_hbm, dinv_hbm, t2_out, a_out,
           src_v, dst_v, g_v, acc_v, tmp_v, sacc_v, dv_v, gs_v, g_sh):
    cid = lax.axis_index("c")
    sid = lax.axis_index("s")
    wid = sid * NC + cid
    base = sid * ZROWS

    pltpu.sync_copy(t1p_hbm.at[:, pl.ds(base, ZROWS)], tmp_v)

    def rbody(i, _):
        sl = pl.ds(i * L, L)
        acc = tmp_v[0, sl]
        for p in range(1, NW):
            acc = acc + tmp_v[p, sl]
        sacc_v[sl] = acc
        return 0

    lax.fori_loop(0, ZROWS // L, rbody, 0)

    pltpu.sync_copy(dinv_hbm.at[pl.ds(sid * SROW, SROW)], dv_v)
    for i in range(SROW):
        dv = dv_v[i]
        a = dv * (sacc_v[pl.ds(i * L, L)] + dv)
        dv_v[i] = a
        gs_v[i] = dv * a
    pltpu.sync_copy(gs_v, g_sh.at[pl.ds(sid * SROW, SROW)])

    @pl.when(cid == 0)
    def _():
        pltpu.sync_copy(dv_v, a_out.at[pl.ds(sid * SROW, SROW)])

    plsc.subcore_barrier()
    pltpu.sync_copy(g_sh, g_v)

    pltpu.sync_copy(src_hbm.at[wid], src_v)
    pltpu.sync_copy(dst_hbm.at[wid], dst_v)
    _zero_1d(acc_v, NPAD // L)

    def body(i, _):
        d_idx = dst_v[pl.ds(i * L, L)]
        s_idx = src_v[pl.ds(i * L, L)]
        vals = plsc.load_gather(g_v, [d_idx >> 4, d_idx & 15])
        plsc.addupdate_scatter(acc_v, [s_idx], vals)
        return 0

    lax.fori_loop(0, EPW_PAD // L, body, 0)
    pltpu.sync_copy(acc_v, t2_out.at[wid])


def _tc_final_body(xw_ref, dinv_ref, a_ref, z0_ref, z1_ref, t2T_ref, b1_ref,
                   w2_ref, w3_ref, wl_ref, b2_ref, b3_ref, bl_ref, out_ref):
    dinv = dinv_ref[...]
    a = a_ref[...]
    y = dinv * xw_ref[...]
    z = z0_ref[...] + z1_ref[...]
    h1 = jnp.maximum(dinv * (z + y) + b1_ref[...], 0.0)
    t2 = jnp.sum(t2T_ref[...], axis=1, keepdims=True)
    w = dinv * t2 + dinv * dinv * a
    u = jnp.sum(w * h1, axis=0, keepdims=True)
    s = jnp.sum(a, axis=0, keepdims=True)
    w3 = w3_ref[...]
    w23 = jnp.dot(w2_ref[...], w3, preferred_element_type=jnp.float32)
    pooled = (jnp.dot(u, w23, preferred_element_type=jnp.float32)
              + s * jnp.dot(b2_ref[...], w3,
                            preferred_element_type=jnp.float32)
              ) * (1.0 / N) + b3_ref[...]
    logit = jnp.dot(pooled, wl_ref[...],
                    preferred_element_type=jnp.float32) + bl_ref[...]
    out_ref[...] = jax.nn.sigmoid(logit)


_tc_final = pl.pallas_call(
    _tc_final_body,
    out_shape=jax.ShapeDtypeStruct((1, 1), jnp.float32),
)


def kernel(x, edge_index, batch, W1, b1, W2, b2, W3, b3, Wl, bl):
    del batch
    src = edge_index[0].astype(jnp.int32).reshape(NW, EPW)
    dst = edge_index[1].astype(jnp.int32).reshape(NW, EPW)
    pad = EPW_PAD - EPW
    src_p = jnp.pad(src, ((0, 0), (0, pad)))
    dst_p = jnp.pad(dst, ((0, 0), (0, pad)), constant_values=N)
    src3 = src_p.reshape(NW, RPW, 128)
    dst3 = dst_p.reshape(NW, RPW, 128)

    xw = _tc_xw(x, W1)
    t1_p, z_p, dinv = _sc_edge(src3, dst3, xw)
    t2_p, a = _sc_t2(src_p, dst_p, t1_p, dinv)
    out = _tc_final(xw, dinv.reshape(NPAD, 1), a.reshape(NPAD, 1),
                    z_p[0], z_p[1], t2_p.T, b1.reshape(1, H),
                    W2, W3, Wl, b2.reshape(1, H), b3.reshape(1, H),
                    bl.reshape(1, 1))
    return out


# speedup vs baseline: 1.1027x; 1.1027x over previous
import functools

import jax
import jax.numpy as jnp
from jax import lax
from jax.experimental import pallas as pl
from jax.experimental.pallas import tpu as pltpu
from jax.experimental.pallas import tpu_sc as plsc

N = 10000
E = 320000
D = 128
H = 16
NC = 2
NS = 16
L = 16
NW = NC * NS
EPW = E // NW
NBUF = 8
RPW = 80
EPW_PAD = RPW * 128
NPAD = RPW * 128
ZROWS = NPAD // NS
OUT = RPW // NBUF

_mesh = plsc.VectorSubcoreMesh(core_axis_name="c", subcore_axis_name="s")
_sc_params = pltpu.CompilerParams(needs_layout_passes=False,
                                  use_tc_tiling_on_sc=False)


def _zero_1d(ref, nvecs):
    zero = jnp.zeros((L,), jnp.float32)

    def body(i, _):
        ref[pl.ds(i * L, L)] = zero
        return 0

    lax.fori_loop(0, nvecs, body, 0)


def _rsqrt16(d):
    i = plsc.bitcast(d, jnp.int32)
    i = jnp.int32(0x5F3759DF) - (i >> 1)
    x = plsc.bitcast(i, jnp.float32)
    for _ in range(3):
        x = x * (1.5 - 0.5 * d * x * x)
    return x


def _tc_xw_body(x_ref, w1_ref, xw_ref):
    xw_ref[:N] = jnp.dot(x_ref[...], w1_ref[...],
                         preferred_element_type=jnp.float32)
    xw_ref[N:] = jnp.zeros((NPAD - N, H), jnp.float32)


_tc_xw = pl.pallas_call(
    _tc_xw_body,
    out_shape=jax.ShapeDtypeStruct((NPAD, H), jnp.float32),
)


@functools.partial(
    pl.kernel,
    out_type=jax.ShapeDtypeStruct((NW, NPAD), jnp.float32),
    mesh=_mesh,
    compiler_params=_sc_params,
    scratch_types=[
        pltpu.VMEM((EPW_PAD,), jnp.int32),
        pltpu.VMEM((NPAD,), jnp.float32),
    ],
)
def _sc_deg(dst_hbm, out_hbm, dst_v, acc_v):
    wid = lax.axis_index("s") * NC + lax.axis_index("c")
    pltpu.sync_copy(dst_hbm.at[wid], dst_v)
    _zero_1d(acc_v, NPAD // L)
    ones = jnp.ones((L,), jnp.float32)

    def body(i, _):
        idx = dst_v[pl.ds(i * L, L)]
        plsc.addupdate_scatter(acc_v, [idx], ones)
        return 0

    lax.fori_loop(0, EPW_PAD // L, body, 0)
    pltpu.sync_copy(acc_v, out_hbm.at[wid])


@functools.partial(
    pl.kernel,
    out_type=(
        jax.ShapeDtypeStruct((NW, NPAD), jnp.float32),
        jax.ShapeDtypeStruct((NC, NPAD, H), jnp.float32),
        jax.ShapeDtypeStruct((NPAD,), jnp.float32),
    ),
    mesh=_mesh,
    compiler_params=_sc_params,
    scratch_types=[
        pltpu.VMEM((RPW, 128), jnp.int32),
        pltpu.VMEM((RPW, 128), jnp.int32),
        pltpu.VMEM((NPAD,), jnp.float32),
        pltpu.VMEM((NPAD,), jnp.float32),
        pltpu.VMEM((NBUF, 128, H), jnp.float32),
        pltpu.VMEM((ZROWS, H), jnp.float32),
        pltpu.VMEM((NW, ZROWS), jnp.float32),
        pltpu.VMEM((ZROWS,), jnp.float32),
        pltpu.VMEM_SHARED((NPAD, H), jnp.float32),
        pltpu.VMEM_SHARED((NPAD, H), jnp.float32),
        pltpu.VMEM_SHARED((NPAD,), jnp.float32),
        pltpu.SemaphoreType.DMA((NBUF,)),
        pltpu.SemaphoreType.DMA((NBUF,)),
    ],
)
def _sc_edge(src3_hbm, dst3_hbm, degp_hbm, xw_hbm,
             t1_out, z_out, dinv_out,
             src_r, dst_r, dinv_v, t1_v, rows_v, sl16_v, tmp_v, dacc_v,
             z_acc, y_sh, dinv_sh, gsem, ssem):
    cid = lax.axis_index("c")
    sid = lax.axis_index("s")
    wid = sid * NC + cid
    base = sid * ZROWS

    zrow = jnp.zeros((L,), jnp.float32)

    def zbody(i, _):
        sl16_v[i] = zrow
        return 0

    lax.fori_loop(0, ZROWS, zbody, 0)
    pltpu.sync_copy(sl16_v, z_acc.at[pl.ds(base, ZROWS)])

    pltpu.sync_copy(degp_hbm.at[:, pl.ds(base, ZROWS)], tmp_v)

    def rbody(i, _):
        sl = pl.ds(i * L, L)
        acc = tmp_v[0, sl]
        for p in range(1, NW):
            acc = acc + tmp_v[p, sl]
        dacc_v[sl] = acc
        return 0

    lax.fori_loop(0, ZROWS // L, rbody, 0)

    iota = lax.iota(jnp.int32, L)
    for i in range(ZROWS // L):
        sl = pl.ds(i * L, L)
        d = dacc_v[sl] + 1.0
        r = _rsqrt16(d)
        mask = (iota + (base + i * L)) < N
        dacc_v[sl] = jnp.where(mask, r, 0.0)
    pltpu.sync_copy(dacc_v, dinv_sh.at[pl.ds(base, ZROWS)])

    @pl.when(cid == 0)
    def _():
        pltpu.sync_copy(dacc_v, dinv_out.at[pl.ds(base, ZROWS)])

    pltpu.sync_copy(xw_hbm.at[pl.ds(base, ZROWS)], sl16_v)

    def ybody(i, _):
        dv = dacc_v[pl.ds(i * L, L)]
        for k in range(L):
            r = i * L + k
            sl16_v[r] = sl16_v[r] * dv[k]
        return 0

    lax.fori_loop(0, ZROWS // L, ybody, 0)
    pltpu.sync_copy(sl16_v, y_sh.at[pl.ds(base, ZROWS)])
    plsc.subcore_barrier()

    pltpu.sync_copy(dinv_sh, dinv_v)
    pltpu.sync_copy(src3_hbm.at[wid], src_r)
    pltpu.sync_copy(dst3_hbm.at[wid], dst_r)
    _zero_1d(t1_v, NPAD // L)

    def _t1_row(j):
        for k in range(128 // L):
            d_idx = dst_r[j, pl.ds(k * L, L)]
            s_idx = src_r[j, pl.ds(k * L, L)]
            vals = plsc.load_gather(dinv_v, [d_idx])
            plsc.addupdate_scatter(t1_v, [s_idx], vals)

    for b in range(NBUF):
        pltpu.async_copy(y_sh.at[src_r.at[b]], rows_v.at[b], gsem.at[b])

    def pipe_body(o, _):
        for b in range(NBUF):
            j = o * NBUF + b
            _t1_row(j)
            pltpu.make_async_copy(
                y_sh.at[src_r.at[j]], rows_v.at[b], gsem.at[b]).wait()
            pltpu.async_copy(rows_v.at[b], z_acc.at[dst_r.at[j]],
                             ssem.at[b], add=True)
            pltpu.make_async_copy(
                rows_v.at[b], z_acc.at[dst_r.at[j]], ssem.at[b]).wait()
            pltpu.async_copy(y_sh.at[src_r.at[j + NBUF]], rows_v.at[b],
                             gsem.at[b])
        return 0

    lax.fori_loop(0, OUT - 1, pipe_body, 0)
    for b in range(NBUF):
        j = (OUT - 1) * NBUF + b
        _t1_row(j)
        pltpu.make_async_copy(
            y_sh.at[src_r.at[j]], rows_v.at[b], gsem.at[b]).wait()
        pltpu.async_copy(rows_v.at[b], z_acc.at[dst_r.at[j]],
                         ssem.at[b], add=True)
        pltpu.make_async_copy(
            rows_v.at[b], z_acc.at[dst_r.at[j]], ssem.at[b]).wait()

    pltpu.sync_copy(t1_v, t1_out.at[wid])
    plsc.subcore_barrier()
    pltpu.sync_copy(z_acc.at[pl.ds(base, ZROWS)], sl16_v)
    pltpu.sync_copy(sl16_v, z_out.at[cid].at[pl.ds(base, ZROWS)])


@functools.partial(
    pl.kernel,
    out_type=(
        jax.ShapeDtypeStruct((NW, NPAD), jnp.float32),
        jax.ShapeDtypeStruct((NPAD,), jnp.float32),
    ),
    mesh=_mesh,
    compiler_params=_sc_params,
    scratch_types=[
        pltpu.VMEM((EPW_PAD,), jnp.int32),
        pltpu.VMEM((EPW_PAD,), jnp.int32),
        pltpu.VMEM((NPAD,), jnp.float32),
        pltpu.VMEM((NPAD,), jnp.float32),
        pltpu.VMEM((NW, ZROWS), jnp.float32),
        pltpu.VMEM((ZROWS,), jnp.float32),
        pltpu.VMEM((ZROWS,), jnp.float32),
        pltpu.VMEM_SHARED((NPAD,), jnp.float32),
    ],
)
def _sc_t2(src_hbm, dst_hbm, t1p_hbm, dinv_hbm, t2_out, a_out,
           src_v, dst_v, g_v, acc_v, tmp_v, sacc_v, dv_v, g_sh):
    cid = lax.axis_index("c")
    sid = lax.axis_index("s")
    wid = sid * NC + cid
    base = sid * ZROWS

    pltpu.sync_copy(t1p_hbm.at[:, pl.ds(base, ZROWS)], tmp_v)

    def rbody(i, _):
        sl = pl.ds(i * L, L)
        acc = tmp_v[0, sl]
        for p in range(1, NW):
            acc = acc + tmp_v[p, sl]
        sacc_v[sl] = acc
        return 0

    lax.fori_loop(0, ZROWS // L, rbody, 0)

    pltpu.sync_copy(dinv_hbm.at[pl.ds(base, ZROWS)], dv_v)
    for i in range(ZROWS // L):
        sl = pl.ds(i * L, L)
        dv = dv_v[sl]
        a = dv * (sacc_v[sl] + dv)
        sacc_v[sl] = a
        dv_v[sl] = dv * a
    pltpu.sync_copy(dv_v, g_sh.at[pl.ds(base, ZROWS)])

    @pl.when(cid == 0)
    def _():
        pltpu.sync_copy(sacc_v, a_out.at[pl.ds(base, ZROWS)])

    plsc.subcore_barrier()
    pltpu.sync_copy(g_sh, g_v)

    pltpu.sync_copy(src_hbm.at[wid], src_v)
    pltpu.sync_copy(dst_hbm.at[wid], dst_v)
    _zero_1d(acc_v, NPAD // L)

    def body(i, _):
        d_idx = dst_v[pl.ds(i * L, L)]
        s_idx = src_v[pl.ds(i * L, L)]
        vals = plsc.load_gather(g_v, [d_idx])
        plsc.addupdate_scatter(acc_v, [s_idx], vals)
        return 0

    lax.fori_loop(0, EPW_PAD // L, body, 0)
    pltpu.sync_copy(acc_v, t2_out.at[wid])


def _tc_final_body(xw_ref, dinv_ref, a_ref, z0_ref, z1_ref, t2T_ref, b1_ref,
                   w2_ref, w3_ref, wl_ref, b2_ref, b3_ref, bl_ref, out_ref):
    dinv = dinv_ref[...]
    a = a_ref[...]
    y = dinv * xw_ref[...]
    z = z0_ref[...] + z1_ref[...]
    h1 = jnp.maximum(dinv * (z + y) + b1_ref[...], 0.0)
    t2 = jnp.sum(t2T_ref[...], axis=1, keepdims=True)
    w = dinv * t2 + dinv * dinv * a
    u = jnp.sum(w * h1, axis=0, keepdims=True)
    s = jnp.sum(a, axis=0, keepdims=True)
    w3 = w3_ref[...]
    w23 = jnp.dot(w2_ref[...], w3, preferred_element_type=jnp.float32)
    pooled = (jnp.dot(u, w23, preferred_element_type=jnp.float32)
              + s * jnp.dot(b2_ref[...], w3,
                            preferred_element_type=jnp.float32)
              ) * (1.0 / N) + b3_ref[...]
    logit = jnp.dot(pooled, wl_ref[...],
                    preferred_element_type=jnp.float32) + bl_ref[...]
    out_ref[...] = jax.nn.sigmoid(logit)


_tc_final = pl.pallas_call(
    _tc_final_body,
    out_shape=jax.ShapeDtypeStruct((1, 1), jnp.float32),
)


def kernel(x, edge_index, batch, W1, b1, W2, b2, W3, b3, Wl, bl):
    del batch
    src = edge_index[0].astype(jnp.int32).reshape(NW, EPW)
    dst = edge_index[1].astype(jnp.int32).reshape(NW, EPW)
    pad = EPW_PAD - EPW
    src_p = jnp.pad(src, ((0, 0), (0, pad)))
    dst_p = jnp.pad(dst, ((0, 0), (0, pad)), constant_values=N)
    src3 = src_p.reshape(NW, RPW, 128)
    dst3 = dst_p.reshape(NW, RPW, 128)

    xw = _tc_xw(x, W1)
    deg_p = _sc_deg(dst_p)
    t1_p, z_p, dinv = _sc_edge(src3, dst3, deg_p, xw)
    t2_p, a = _sc_t2(src_p, dst_p, t1_p, dinv)
    out = _tc_final(xw, dinv.reshape(NPAD, 1), a.reshape(NPAD, 1),
                    z_p[0], z_p[1], t2_p.T, b1.reshape(1, H),
                    W2, W3, Wl, b2.reshape(1, H), b3.reshape(1, H),
                    bl.reshape(1, 1))
    return out


# speedup vs baseline: 1.1357x; 1.0299x over previous
import functools

import jax
import jax.numpy as jnp
from jax import lax
from jax.experimental import pallas as pl
from jax.experimental.pallas import tpu as pltpu
from jax.experimental.pallas import tpu_sc as plsc

N = 10000
E = 320000
D = 128
H = 16
NC = 2
NS = 16
L = 16
NW = NC * NS
EPW = E // NW
NBUF = 8
RPW = 80
EPW_PAD = RPW * 128
NPAD = RPW * 128
ZROWS = NPAD // NS
OUT = RPW // NBUF

_mesh = plsc.VectorSubcoreMesh(core_axis_name="c", subcore_axis_name="s")
_sc_params = pltpu.CompilerParams(needs_layout_passes=False,
                                  use_tc_tiling_on_sc=False)


def _zero_1d(ref, nvecs):
    zero = jnp.zeros((L,), jnp.float32)

    def body(i, _):
        ref[pl.ds(i * L, L)] = zero
        return 0

    lax.fori_loop(0, nvecs, body, 0)


def _rsqrt16(d):
    i = plsc.bitcast(d, jnp.int32)
    i = jnp.int32(0x5F3759DF) - (i >> 1)
    x = plsc.bitcast(i, jnp.float32)
    for _ in range(3):
        x = x * (1.5 - 0.5 * d * x * x)
    return x


def _tc_xw_body(x_ref, w1_ref, xw_ref):
    xw_ref[:N] = jnp.dot(x_ref[...], w1_ref[...],
                         preferred_element_type=jnp.float32)
    xw_ref[N:] = jnp.zeros((NPAD - N, H), jnp.float32)


_tc_xw = pl.pallas_call(
    _tc_xw_body,
    out_shape=jax.ShapeDtypeStruct((NPAD, H), jnp.float32),
)


@functools.partial(
    pl.kernel,
    out_type=jax.ShapeDtypeStruct((NW, NPAD), jnp.float32),
    mesh=_mesh,
    compiler_params=_sc_params,
    scratch_types=[
        pltpu.VMEM((EPW_PAD,), jnp.int32),
        pltpu.VMEM((NPAD,), jnp.float32),
    ],
)
def _sc_deg(dst_hbm, out_hbm, dst_v, acc_v):
    wid = lax.axis_index("s") * NC + lax.axis_index("c")
    pltpu.sync_copy(dst_hbm.at[wid], dst_v)
    _zero_1d(acc_v, NPAD // L)
    ones = jnp.ones((L,), jnp.float32)

    def body(i, _):
        idx = dst_v[pl.ds(i * L, L)]
        plsc.addupdate_scatter(acc_v, [idx], ones)
        return 0

    lax.fori_loop(0, EPW_PAD // L, body, 0)
    pltpu.sync_copy(acc_v, out_hbm.at[wid])


@functools.partial(
    pl.kernel,
    out_type=(
        jax.ShapeDtypeStruct((NW, NPAD), jnp.float32),
        jax.ShapeDtypeStruct((NC, NPAD, H), jnp.float32),
        jax.ShapeDtypeStruct((NPAD,), jnp.float32),
    ),
    mesh=_mesh,
    compiler_params=_sc_params,
    scratch_types=[
        pltpu.VMEM((RPW, 128), jnp.int32),
        pltpu.VMEM((RPW, 128), jnp.int32),
        pltpu.VMEM((NPAD,), jnp.float32),
        pltpu.VMEM((NPAD,), jnp.float32),
        pltpu.VMEM((NBUF, 128, H), jnp.float32),
        pltpu.VMEM((ZROWS, H), jnp.float32),
        pltpu.VMEM((NW, ZROWS), jnp.float32),
        pltpu.VMEM((ZROWS,), jnp.float32),
        pltpu.VMEM_SHARED((NPAD, H), jnp.float32),
        pltpu.VMEM_SHARED((NPAD, H), jnp.float32),
        pltpu.VMEM_SHARED((NPAD,), jnp.float32),
        pltpu.SemaphoreType.DMA((NBUF,)),
        pltpu.SemaphoreType.DMA((NBUF,)),
        pltpu.SemaphoreType.DMA,
        pltpu.SemaphoreType.DMA,
    ],
)
def _sc_edge(src3_hbm, dst3_hbm, degp_hbm, xw_hbm,
             t1_out, z_out, dinv_out,
             src_r, dst_r, dinv_v, t1_v, rows_v, sl16_v, tmp_v, dacc_v,
             z_acc, y_sh, dinv_sh, gsem, ssem, s_sem, d_sem):
    cid = lax.axis_index("c")
    sid = lax.axis_index("s")
    wid = sid * NC + cid
    base = sid * ZROWS

    src_cp = pltpu.async_copy(src3_hbm.at[wid], src_r, s_sem)
    dst_cp = pltpu.async_copy(dst3_hbm.at[wid], dst_r, d_sem)

    zrow = jnp.zeros((L,), jnp.float32)

    def zbody(i, _):
        sl16_v[i] = zrow
        return 0

    lax.fori_loop(0, ZROWS, zbody, 0)
    pltpu.sync_copy(sl16_v, z_acc.at[pl.ds(base, ZROWS)])

    pltpu.sync_copy(degp_hbm.at[:, pl.ds(base, ZROWS)], tmp_v)

    def rbody(i, _):
        sl = pl.ds(i * L, L)
        acc = tmp_v[0, sl]
        for p in range(1, NW):
            acc = acc + tmp_v[p, sl]
        dacc_v[sl] = acc
        return 0

    lax.fori_loop(0, ZROWS // L, rbody, 0)

    iota = lax.iota(jnp.int32, L)
    for i in range(ZROWS // L):
        sl = pl.ds(i * L, L)
        d = dacc_v[sl] + 1.0
        r = _rsqrt16(d)
        mask = (iota + (base + i * L)) < N
        dacc_v[sl] = jnp.where(mask, r, 0.0)
    pltpu.sync_copy(dacc_v, dinv_sh.at[pl.ds(base, ZROWS)])

    @pl.when(cid == 0)
    def _():
        pltpu.sync_copy(dacc_v, dinv_out.at[pl.ds(base, ZROWS)])

    pltpu.sync_copy(xw_hbm.at[pl.ds(base, ZROWS)], sl16_v)

    def ybody(i, _):
        dv = dacc_v[pl.ds(i * L, L)]
        for k in range(L):
            r = i * L + k
            sl16_v[r] = sl16_v[r] * dv[k]
        return 0

    lax.fori_loop(0, ZROWS // L, ybody, 0)
    pltpu.sync_copy(sl16_v, y_sh.at[pl.ds(base, ZROWS)])
    plsc.subcore_barrier()

    pltpu.sync_copy(dinv_sh, dinv_v)
    src_cp.wait()
    dst_cp.wait()
    _zero_1d(t1_v, NPAD // L)

    def _t1_row(j):
        for k in range(128 // L):
            d_idx = dst_r[j, pl.ds(k * L, L)]
            s_idx = src_r[j, pl.ds(k * L, L)]
            vals = plsc.load_gather(dinv_v, [d_idx])
            plsc.addupdate_scatter(t1_v, [s_idx], vals)

    for b in range(NBUF):
        pltpu.async_copy(y_sh.at[src_r.at[b]], rows_v.at[b], gsem.at[b])

    def pipe_body(o, _):
        for b in range(NBUF):
            j = o * NBUF + b
            _t1_row(j)
            pltpu.make_async_copy(
                y_sh.at[src_r.at[j]], rows_v.at[b], gsem.at[b]).wait()
            pltpu.async_copy(rows_v.at[b], z_acc.at[dst_r.at[j]],
                             ssem.at[b], add=True)
            pltpu.make_async_copy(
                rows_v.at[b], z_acc.at[dst_r.at[j]], ssem.at[b]).wait()
            pltpu.async_copy(y_sh.at[src_r.at[j + NBUF]], rows_v.at[b],
                             gsem.at[b])
        return 0

    lax.fori_loop(0, OUT - 1, pipe_body, 0)
    for b in range(NBUF):
        j = (OUT - 1) * NBUF + b
        _t1_row(j)
        pltpu.make_async_copy(
            y_sh.at[src_r.at[j]], rows_v.at[b], gsem.at[b]).wait()
        pltpu.async_copy(rows_v.at[b], z_acc.at[dst_r.at[j]],
                         ssem.at[b], add=True)
        pltpu.make_async_copy(
            rows_v.at[b], z_acc.at[dst_r.at[j]], ssem.at[b]).wait()

    pltpu.sync_copy(t1_v, t1_out.at[wid])
    plsc.subcore_barrier()
    pltpu.sync_copy(z_acc.at[pl.ds(base, ZROWS)], sl16_v)
    pltpu.sync_copy(sl16_v, z_out.at[cid].at[pl.ds(base, ZROWS)])


@functools.partial(
    pl.kernel,
    out_type=(
        jax.ShapeDtypeStruct((NW, NPAD), jnp.float32),
        jax.ShapeDtypeStruct((NPAD,), jnp.float32),
    ),
    mesh=_mesh,
    compiler_params=_sc_params,
    scratch_types=[
        pltpu.VMEM((EPW_PAD,), jnp.int32),
        pltpu.VMEM((EPW_PAD,), jnp.int32),
        pltpu.VMEM((NPAD,), jnp.float32),
        pltpu.VMEM((NPAD,), jnp.float32),
        pltpu.VMEM((NW, ZROWS), jnp.float32),
        pltpu.VMEM((ZROWS,), jnp.float32),
        pltpu.VMEM((ZROWS,), jnp.float32),
        pltpu.VMEM_SHARED((NPAD,), jnp.float32),
        pltpu.SemaphoreType.DMA,
        pltpu.SemaphoreType.DMA,
    ],
)
def _sc_t2(src_hbm, dst_hbm, t1p_hbm, dinv_hbm, t2_out, a_out,
           src_v, dst_v, g_v, acc_v, tmp_v, sacc_v, dv_v, g_sh,
           s_sem, d_sem):
    cid = lax.axis_index("c")
    sid = lax.axis_index("s")
    wid = sid * NC + cid
    base = sid * ZROWS

    src_cp = pltpu.async_copy(src_hbm.at[wid], src_v, s_sem)
    dst_cp = pltpu.async_copy(dst_hbm.at[wid], dst_v, d_sem)
    pltpu.sync_copy(t1p_hbm.at[:, pl.ds(base, ZROWS)], tmp_v)

    def rbody(i, _):
        sl = pl.ds(i * L, L)
        acc = tmp_v[0, sl]
        for p in range(1, NW):
            acc = acc + tmp_v[p, sl]
        sacc_v[sl] = acc
        return 0

    lax.fori_loop(0, ZROWS // L, rbody, 0)

    pltpu.sync_copy(dinv_hbm.at[pl.ds(base, ZROWS)], dv_v)
    for i in range(ZROWS // L):
        sl = pl.ds(i * L, L)
        dv = dv_v[sl]
        a = dv * (sacc_v[sl] + dv)
        sacc_v[sl] = a
        dv_v[sl] = dv * a
    pltpu.sync_copy(dv_v, g_sh.at[pl.ds(base, ZROWS)])

    @pl.when(cid == 0)
    def _():
        pltpu.sync_copy(sacc_v, a_out.at[pl.ds(base, ZROWS)])

    plsc.subcore_barrier()
    pltpu.sync_copy(g_sh, g_v)

    src_cp.wait()
    dst_cp.wait()
    _zero_1d(acc_v, NPAD // L)

    def body(i, _):
        d_idx = dst_v[pl.ds(i * L, L)]
        s_idx = src_v[pl.ds(i * L, L)]
        vals = plsc.load_gather(g_v, [d_idx])
        plsc.addupdate_scatter(acc_v, [s_idx], vals)
        return 0

    lax.fori_loop(0, EPW_PAD // L, body, 0)
    pltpu.sync_copy(acc_v, t2_out.at[wid])


def _tc_final_body(xw_ref, dinv_ref, a_ref, z0_ref, z1_ref, t2T_ref, b1_ref,
                   w2_ref, w3_ref, wl_ref, b2_ref, b3_ref, bl_ref, out_ref):
    dinv = dinv_ref[...]
    a = a_ref[...]
    y = dinv * xw_ref[...]
    z = z0_ref[...] + z1_ref[...]
    h1 = jnp.maximum(dinv * (z + y) + b1_ref[...], 0.0)
    t2 = jnp.sum(t2T_ref[...], axis=1, keepdims=True)
    w = dinv * t2 + dinv * dinv * a
    u = jnp.sum(w * h1, axis=0, keepdims=True)
    s = jnp.sum(a, axis=0, keepdims=True)
    w3 = w3_ref[...]
    w23 = jnp.dot(w2_ref[...], w3, preferred_element_type=jnp.float32)
    pooled = (jnp.dot(u, w23, preferred_element_type=jnp.float32)
              + s * jnp.dot(b2_ref[...], w3,
                            preferred_element_type=jnp.float32)
              ) * (1.0 / N) + b3_ref[...]
    logit = jnp.dot(pooled, wl_ref[...],
                    preferred_element_type=jnp.float32) + bl_ref[...]
    out_ref[...] = jax.nn.sigmoid(logit)


_tc_final = pl.pallas_call(
    _tc_final_body,
    out_shape=jax.ShapeDtypeStruct((1, 1), jnp.float32),
)


def kernel(x, edge_index, batch, W1, b1, W2, b2, W3, b3, Wl, bl):
    del batch
    src = edge_index[0].astype(jnp.int32).reshape(NW, EPW)
    dst = edge_index[1].astype(jnp.int32).reshape(NW, EPW)
    pad = EPW_PAD - EPW
    src_p = jnp.pad(src, ((0, 0), (0, pad)))
    dst_p = jnp.pad(dst, ((0, 0), (0, pad)), constant_values=N)
    src3 = src_p.reshape(NW, RPW, 128)
    dst3 = dst_p.reshape(NW, RPW, 128)

    xw = _tc_xw(x, W1)
    deg_p = _sc_deg(dst_p)
    t1_p, z_p, dinv = _sc_edge(src3, dst3, deg_p, xw)
    t2_p, a = _sc_t2(src_p, dst_p, t1_p, dinv)
    out = _tc_final(xw, dinv.reshape(NPAD, 1), a.reshape(NPAD, 1),
                    z_p[0], z_p[1], t2_p.T, b1.reshape(1, H),
                    W2, W3, Wl, b2.reshape(1, H), b3.reshape(1, H),
                    bl.reshape(1, 1))
    return out


# speedup vs baseline: 1.1776x; 1.0369x over previous
import functools

import jax
import jax.numpy as jnp
from jax import lax
from jax.experimental import pallas as pl
from jax.experimental.pallas import tpu as pltpu
from jax.experimental.pallas import tpu_sc as plsc

N = 10000
E = 320000
D = 128
H = 16
NC = 2
NS = 16
L = 16
NW = NC * NS
EPW = E // NW
NBUF = 8
RPW = 80
EPW_PAD = RPW * 128
NPAD = RPW * 128
ZROWS = NPAD // NS
OUT = RPW // NBUF

_mesh = plsc.VectorSubcoreMesh(core_axis_name="c", subcore_axis_name="s")
_sc_params = pltpu.CompilerParams(needs_layout_passes=False,
                                  use_tc_tiling_on_sc=False)


def _zero_1d(ref, nvecs):
    zero = jnp.zeros((L,), jnp.float32)

    def body(i, _):
        ref[pl.ds(i * L, L)] = zero
        return 0

    lax.fori_loop(0, nvecs, body, 0)


def _rsqrt16(d):
    i = plsc.bitcast(d, jnp.int32)
    i = jnp.int32(0x5F3759DF) - (i >> 1)
    x = plsc.bitcast(i, jnp.float32)
    for _ in range(3):
        x = x * (1.5 - 0.5 * d * x * x)
    return x


def _tc_xw_body(x_ref, w1_ref, xw_ref):
    xw_ref[:N] = jnp.dot(x_ref[...], w1_ref[...],
                         preferred_element_type=jnp.float32)
    xw_ref[N:] = jnp.zeros((NPAD - N, H), jnp.float32)


_tc_xw = pl.pallas_call(
    _tc_xw_body,
    out_shape=jax.ShapeDtypeStruct((NPAD, H), jnp.float32),
)


@functools.partial(
    pl.kernel,
    out_type=jax.ShapeDtypeStruct((NW, NPAD), jnp.float32),
    mesh=_mesh,
    compiler_params=_sc_params,
    scratch_types=[
        pltpu.VMEM((EPW_PAD,), jnp.int32),
        pltpu.VMEM((NPAD,), jnp.float32),
    ],
)
def _sc_deg(dst_hbm, out_hbm, dst_v, acc_v):
    wid = lax.axis_index("s") * NC + lax.axis_index("c")
    pltpu.sync_copy(dst_hbm.at[wid], dst_v)
    _zero_1d(acc_v, NPAD // L)
    ones = jnp.ones((L,), jnp.float32)

    def body(i, _):
        idx = dst_v[pl.ds(i * L, L)]
        plsc.addupdate_scatter(acc_v, [idx], ones)
        return 0

    lax.fori_loop(0, EPW_PAD // L, body, 0)
    pltpu.sync_copy(acc_v, out_hbm.at[wid])


@functools.partial(
    pl.kernel,
    out_type=(
        jax.ShapeDtypeStruct((NW, NPAD), jnp.float32),
        jax.ShapeDtypeStruct((NC, NPAD, H), jnp.float32),
        jax.ShapeDtypeStruct((NPAD,), jnp.float32),
    ),
    mesh=_mesh,
    compiler_params=_sc_params,
    scratch_types=[
        pltpu.VMEM((RPW, 128), jnp.int32),
        pltpu.VMEM((RPW, 128), jnp.int32),
        pltpu.VMEM((NPAD,), jnp.float32),
        pltpu.VMEM((NPAD,), jnp.float32),
        pltpu.VMEM((NBUF, 128, H), jnp.float32),
        pltpu.VMEM((ZROWS, H), jnp.float32),
        pltpu.VMEM((NW, ZROWS), jnp.float32),
        pltpu.VMEM((ZROWS,), jnp.float32),
        pltpu.VMEM_SHARED((NPAD, H), jnp.float32),
        pltpu.VMEM_SHARED((NPAD, H), jnp.float32),
        pltpu.VMEM_SHARED((NPAD,), jnp.float32),
        pltpu.SemaphoreType.DMA((NBUF,)),
        pltpu.SemaphoreType.DMA((NBUF,)),
        pltpu.SemaphoreType.DMA,
        pltpu.SemaphoreType.DMA,
        pltpu.SemaphoreType.DMA,
    ],
)
def _sc_edge(src3_hbm, dst3_hbm, degp_hbm, xw_hbm,
             t1_out, z_out, dinv_out,
             src_r, dst_r, dinv_v, t1_v, rows_v, sl16_v, tmp_v, dacc_v,
             z_acc, y_sh, dinv_sh, gsem, ssem, s_sem, d_sem, p_sem):
    cid = lax.axis_index("c")
    sid = lax.axis_index("s")
    wid = sid * NC + cid
    base = sid * ZROWS

    src_cp = pltpu.async_copy(src3_hbm.at[wid], src_r, s_sem)
    dst_cp = pltpu.async_copy(dst3_hbm.at[wid], dst_r, d_sem)
    deg_cp = pltpu.async_copy(degp_hbm.at[:, pl.ds(base, ZROWS)], tmp_v,
                              p_sem)

    zrow = jnp.zeros((L,), jnp.float32)

    def zbody(i, _):
        sl16_v[i] = zrow
        return 0

    lax.fori_loop(0, ZROWS, zbody, 0)
    pltpu.sync_copy(sl16_v, z_acc.at[pl.ds(base, ZROWS)])
    _zero_1d(t1_v, NPAD // L)

    deg_cp.wait()

    def rbody(i, _):
        sl = pl.ds(i * L, L)
        acc = tmp_v[0, sl]
        for p in range(1, NW):
            acc = acc + tmp_v[p, sl]
        dacc_v[sl] = acc
        return 0

    lax.fori_loop(0, ZROWS // L, rbody, 0)

    iota = lax.iota(jnp.int32, L)
    for i in range(ZROWS // L):
        sl = pl.ds(i * L, L)
        d = dacc_v[sl] + 1.0
        r = _rsqrt16(d)
        mask = (iota + (base + i * L)) < N
        dacc_v[sl] = jnp.where(mask, r, 0.0)
    pltpu.sync_copy(dacc_v, dinv_sh.at[pl.ds(base, ZROWS)])

    @pl.when(cid == 0)
    def _():
        pltpu.sync_copy(dacc_v, dinv_out.at[pl.ds(base, ZROWS)])

    pltpu.sync_copy(xw_hbm.at[pl.ds(base, ZROWS)], sl16_v)

    def ybody(i, _):
        dv = dacc_v[pl.ds(i * L, L)]
        for k in range(L):
            r = i * L + k
            sl16_v[r] = sl16_v[r] * dv[k]
        return 0

    lax.fori_loop(0, ZROWS // L, ybody, 0)
    pltpu.sync_copy(sl16_v, y_sh.at[pl.ds(base, ZROWS)])
    plsc.subcore_barrier()

    pltpu.sync_copy(dinv_sh, dinv_v)
    src_cp.wait()
    dst_cp.wait()

    def _t1_row(j):
        for k in range(128 // L):
            d_idx = dst_r[j, pl.ds(k * L, L)]
            s_idx = src_r[j, pl.ds(k * L, L)]
            vals = plsc.load_gather(dinv_v, [d_idx])
            plsc.addupdate_scatter(t1_v, [s_idx], vals)

    for b in range(NBUF):
        pltpu.async_copy(y_sh.at[src_r.at[b]], rows_v.at[b], gsem.at[b])

    def pipe_body(o, _):
        for b in range(NBUF):
            j = o * NBUF + b
            _t1_row(j)
            pltpu.make_async_copy(
                y_sh.at[src_r.at[j]], rows_v.at[b], gsem.at[b]).wait()
            pltpu.async_copy(rows_v.at[b], z_acc.at[dst_r.at[j]],
                             ssem.at[b], add=True)
            pltpu.make_async_copy(
                rows_v.at[b], z_acc.at[dst_r.at[j]], ssem.at[b]).wait()
            pltpu.async_copy(y_sh.at[src_r.at[j + NBUF]], rows_v.at[b],
                             gsem.at[b])
        return 0

    lax.fori_loop(0, OUT - 1, pipe_body, 0)
    for b in range(NBUF):
        j = (OUT - 1) * NBUF + b
        _t1_row(j)
        pltpu.make_async_copy(
            y_sh.at[src_r.at[j]], rows_v.at[b], gsem.at[b]).wait()
        pltpu.async_copy(rows_v.at[b], z_acc.at[dst_r.at[j]],
                         ssem.at[b], add=True)
        pltpu.make_async_copy(
            rows_v.at[b], z_acc.at[dst_r.at[j]], ssem.at[b]).wait()

    pltpu.sync_copy(t1_v, t1_out.at[wid])
    plsc.subcore_barrier()
    pltpu.sync_copy(z_acc.at[pl.ds(base, ZROWS)], sl16_v)
    pltpu.sync_copy(sl16_v, z_out.at[cid].at[pl.ds(base, ZROWS)])


@functools.partial(
    pl.kernel,
    out_type=(
        jax.ShapeDtypeStruct((NW, NPAD), jnp.float32),
        jax.ShapeDtypeStruct((NPAD,), jnp.float32),
    ),
    mesh=_mesh,
    compiler_params=_sc_params,
    scratch_types=[
        pltpu.VMEM((EPW_PAD,), jnp.int32),
        pltpu.VMEM((EPW_PAD,), jnp.int32),
        pltpu.VMEM((NPAD,), jnp.float32),
        pltpu.VMEM((NPAD,), jnp.float32),
        pltpu.VMEM((NW, ZROWS), jnp.float32),
        pltpu.VMEM((ZROWS,), jnp.float32),
        pltpu.VMEM((ZROWS,), jnp.float32),
        pltpu.VMEM_SHARED((NPAD,), jnp.float32),
        pltpu.SemaphoreType.DMA,
        pltpu.SemaphoreType.DMA,
        pltpu.SemaphoreType.DMA,
    ],
)
def _sc_t2(src_hbm, dst_hbm, t1p_hbm, dinv_hbm, t2_out, a_out,
           src_v, dst_v, g_v, acc_v, tmp_v, sacc_v, dv_v, g_sh,
           s_sem, d_sem, p_sem):
    cid = lax.axis_index("c")
    sid = lax.axis_index("s")
    wid = sid * NC + cid
    base = sid * ZROWS

    src_cp = pltpu.async_copy(src_hbm.at[wid], src_v, s_sem)
    dst_cp = pltpu.async_copy(dst_hbm.at[wid], dst_v, d_sem)
    t1_cp = pltpu.async_copy(t1p_hbm.at[:, pl.ds(base, ZROWS)], tmp_v,
                             p_sem)
    _zero_1d(acc_v, NPAD // L)
    t1_cp.wait()

    def rbody(i, _):
        sl = pl.ds(i * L, L)
        acc = tmp_v[0, sl]
        for p in range(1, NW):
            acc = acc + tmp_v[p, sl]
        sacc_v[sl] = acc
        return 0

    lax.fori_loop(0, ZROWS // L, rbody, 0)

    pltpu.sync_copy(dinv_hbm.at[pl.ds(base, ZROWS)], dv_v)
    for i in range(ZROWS // L):
        sl = pl.ds(i * L, L)
        dv = dv_v[sl]
        a = dv * (sacc_v[sl] + dv)
        sacc_v[sl] = a
        dv_v[sl] = dv * a
    pltpu.sync_copy(dv_v, g_sh.at[pl.ds(base, ZROWS)])

    @pl.when(cid == 0)
    def _():
        pltpu.sync_copy(sacc_v, a_out.at[pl.ds(base, ZROWS)])

    plsc.subcore_barrier()
    pltpu.sync_copy(g_sh, g_v)

    src_cp.wait()
    dst_cp.wait()

    def body(i, _):
        d_idx = dst_v[pl.ds(i * L, L)]
        s_idx = src_v[pl.ds(i * L, L)]
        vals = plsc.load_gather(g_v, [d_idx])
        plsc.addupdate_scatter(acc_v, [s_idx], vals)
        return 0

    lax.fori_loop(0, EPW_PAD // L, body, 0)
    pltpu.sync_copy(acc_v, t2_out.at[wid])


def _tc_final_body(xw_ref, dinv_ref, a_ref, z0_ref, z1_ref, t2T_ref, b1_ref,
                   w2_ref, w3_ref, wl_ref, b2_ref, b3_ref, bl_ref, out_ref):
    dinv = dinv_ref[...]
    a = a_ref[...]
    y = dinv * xw_ref[...]
    z = z0_ref[...] + z1_ref[...]
    h1 = jnp.maximum(dinv * (z + y) + b1_ref[...], 0.0)
    t2 = jnp.sum(t2T_ref[...], axis=1, keepdims=True)
    w = dinv * t2 + dinv * dinv * a
    u = jnp.sum(w * h1, axis=0, keepdims=True)
    s = jnp.sum(a, axis=0, keepdims=True)
    w3 = w3_ref[...]
    w23 = jnp.dot(w2_ref[...], w3, preferred_element_type=jnp.float32)
    pooled = (jnp.dot(u, w23, preferred_element_type=jnp.float32)
              + s * jnp.dot(b2_ref[...], w3,
                            preferred_element_type=jnp.float32)
              ) * (1.0 / N) + b3_ref[...]
    logit = jnp.dot(pooled, wl_ref[...],
                    preferred_element_type=jnp.float32) + bl_ref[...]
    out_ref[...] = jax.nn.sigmoid(logit)


_tc_final = pl.pallas_call(
    _tc_final_body,
    out_shape=jax.ShapeDtypeStruct((1, 1), jnp.float32),
)


def kernel(x, edge_index, batch, W1, b1, W2, b2, W3, b3, Wl, bl):
    del batch
    src = edge_index[0].astype(jnp.int32).reshape(NW, EPW)
    dst = edge_index[1].astype(jnp.int32).reshape(NW, EPW)
    pad = EPW_PAD - EPW
    src_p = jnp.pad(src, ((0, 0), (0, pad)))
    dst_p = jnp.pad(dst, ((0, 0), (0, pad)), constant_values=N)
    src3 = src_p.reshape(NW, RPW, 128)
    dst3 = dst_p.reshape(NW, RPW, 128)

    xw = _tc_xw(x, W1)
    deg_p = _sc_deg(dst_p)
    t1_p, z_p, dinv = _sc_edge(src3, dst3, deg_p, xw)
    t2_p, a = _sc_t2(src_p, dst_p, t1_p, dinv)
    out = _tc_final(xw, dinv.reshape(NPAD, 1), a.reshape(NPAD, 1),
                    z_p[0], z_p[1], t2_p.T, b1.reshape(1, H),
                    W2, W3, Wl, b2.reshape(1, H), b3.reshape(1, H),
                    bl.reshape(1, 1))
    return out


# speedup vs baseline: 1.1827x; 1.0044x over previous
import functools

import jax
import jax.numpy as jnp
from jax import lax
from jax.experimental import pallas as pl
from jax.experimental.pallas import tpu as pltpu
from jax.experimental.pallas import tpu_sc as plsc

N = 10000
E = 320000
D = 128
H = 16
NC = 2
NS = 16
L = 16
NW = NC * NS
EPW = E // NW
NBUF = 8
RPW = 80
EPW_PAD = RPW * 128
NPAD = RPW * 128
ZROWS = NPAD // NS
OUT = RPW // NBUF

_mesh = plsc.VectorSubcoreMesh(core_axis_name="c", subcore_axis_name="s")
_sc_params = pltpu.CompilerParams(needs_layout_passes=False,
                                  use_tc_tiling_on_sc=False)


def _zero_1d(ref, nvecs):
    zero = jnp.zeros((L,), jnp.float32)

    def body(i, _):
        ref[pl.ds(i * L, L)] = zero
        return 0

    lax.fori_loop(0, nvecs, body, 0)


def _rsqrt16(d):
    i = plsc.bitcast(d, jnp.int32)
    i = jnp.int32(0x5F3759DF) - (i >> 1)
    x = plsc.bitcast(i, jnp.float32)
    for _ in range(3):
        x = x * (1.5 - 0.5 * d * x * x)
    return x


def _tc_xw_body(x_ref, w1_ref, xw_ref):
    xw_ref[:N] = jnp.dot(x_ref[...], w1_ref[...],
                         preferred_element_type=jnp.float32)
    xw_ref[N:] = jnp.zeros((NPAD - N, H), jnp.float32)


_tc_xw = pl.pallas_call(
    _tc_xw_body,
    out_shape=jax.ShapeDtypeStruct((NPAD, H), jnp.float32),
)


@functools.partial(
    pl.kernel,
    out_type=jax.ShapeDtypeStruct((NW, NPAD), jnp.float32),
    mesh=_mesh,
    compiler_params=_sc_params,
    scratch_types=[
        pltpu.VMEM((EPW_PAD,), jnp.int32),
        pltpu.VMEM((NPAD,), jnp.float32),
        pltpu.SemaphoreType.DMA,
        pltpu.SemaphoreType.DMA,
    ],
)
def _sc_deg(dst_hbm, out_hbm, dst_v, acc_v, sem0, sem1):
    wid = lax.axis_index("s") * NC + lax.axis_index("c")
    half = EPW_PAD // 2
    cp0 = pltpu.async_copy(dst_hbm.at[wid].at[pl.ds(0, half)],
                           dst_v.at[pl.ds(0, half)], sem0)
    cp1 = pltpu.async_copy(dst_hbm.at[wid].at[pl.ds(half, half)],
                           dst_v.at[pl.ds(half, half)], sem1)
    _zero_1d(acc_v, NPAD // L)
    ones = jnp.ones((L,), jnp.float32)

    def body(i, _):
        idx = dst_v[pl.ds(i * L, L)]
        plsc.addupdate_scatter(acc_v, [idx], ones)
        return 0

    cp0.wait()
    lax.fori_loop(0, half // L, body, 0)
    cp1.wait()
    lax.fori_loop(half // L, EPW_PAD // L, body, 0)
    pltpu.sync_copy(acc_v, out_hbm.at[wid])


@functools.partial(
    pl.kernel,
    out_type=(
        jax.ShapeDtypeStruct((NW, NPAD), jnp.float32),
        jax.ShapeDtypeStruct((NC, NPAD, H), jnp.float32),
        jax.ShapeDtypeStruct((NPAD,), jnp.float32),
    ),
    mesh=_mesh,
    compiler_params=_sc_params,
    scratch_types=[
        pltpu.VMEM((RPW, 128), jnp.int32),
        pltpu.VMEM((RPW, 128), jnp.int32),
        pltpu.VMEM((NPAD,), jnp.float32),
        pltpu.VMEM((NPAD,), jnp.float32),
        pltpu.VMEM((NBUF, 128, H), jnp.float32),
        pltpu.VMEM((ZROWS, H), jnp.float32),
        pltpu.VMEM((NW, ZROWS), jnp.float32),
        pltpu.VMEM((ZROWS,), jnp.float32),
        pltpu.VMEM_SHARED((NPAD, H), jnp.float32),
        pltpu.VMEM_SHARED((NPAD, H), jnp.float32),
        pltpu.VMEM_SHARED((NPAD,), jnp.float32),
        pltpu.SemaphoreType.DMA((NBUF,)),
        pltpu.SemaphoreType.DMA((NBUF,)),
        pltpu.SemaphoreType.DMA,
        pltpu.SemaphoreType.DMA,
        pltpu.SemaphoreType.DMA,
    ],
)
def _sc_edge(src3_hbm, dst3_hbm, degp_hbm, xw_hbm,
             t1_out, z_out, dinv_out,
             src_r, dst_r, dinv_v, t1_v, rows_v, sl16_v, tmp_v, dacc_v,
             z_acc, y_sh, dinv_sh, gsem, ssem, s_sem, d_sem, p_sem):
    cid = lax.axis_index("c")
    sid = lax.axis_index("s")
    wid = sid * NC + cid
    base = sid * ZROWS

    src_cp = pltpu.async_copy(src3_hbm.at[wid], src_r, s_sem)
    dst_cp = pltpu.async_copy(dst3_hbm.at[wid], dst_r, d_sem)
    deg_cp = pltpu.async_copy(degp_hbm.at[:, pl.ds(base, ZROWS)], tmp_v,
                              p_sem)

    zrow = jnp.zeros((L,), jnp.float32)

    def zbody(i, _):
        sl16_v[i] = zrow
        return 0

    lax.fori_loop(0, ZROWS, zbody, 0)
    pltpu.sync_copy(sl16_v, z_acc.at[pl.ds(base, ZROWS)])
    _zero_1d(t1_v, NPAD // L)

    deg_cp.wait()

    def rbody(i, _):
        sl = pl.ds(i * L, L)
        acc = tmp_v[0, sl]
        for p in range(1, NW):
            acc = acc + tmp_v[p, sl]
        dacc_v[sl] = acc
        return 0

    lax.fori_loop(0, ZROWS // L, rbody, 0)

    iota = lax.iota(jnp.int32, L)
    for i in range(ZROWS // L):
        sl = pl.ds(i * L, L)
        d = dacc_v[sl] + 1.0
        r = _rsqrt16(d)
        mask = (iota + (base + i * L)) < N
        dacc_v[sl] = jnp.where(mask, r, 0.0)
    pltpu.sync_copy(dacc_v, dinv_sh.at[pl.ds(base, ZROWS)])

    @pl.when(cid == 0)
    def _():
        pltpu.sync_copy(dacc_v, dinv_out.at[pl.ds(base, ZROWS)])

    pltpu.sync_copy(xw_hbm.at[pl.ds(base, ZROWS)], sl16_v)

    def ybody(i, _):
        dv = dacc_v[pl.ds(i * L, L)]
        for k in range(L):
            r = i * L + k
            sl16_v[r] = sl16_v[r] * dv[k]
        return 0

    lax.fori_loop(0, ZROWS // L, ybody, 0)
    pltpu.sync_copy(sl16_v, y_sh.at[pl.ds(base, ZROWS)])
    plsc.subcore_barrier()

    pltpu.sync_copy(dinv_sh, dinv_v)
    src_cp.wait()
    dst_cp.wait()

    def _t1_row(j):
        for k in range(128 // L):
            d_idx = dst_r[j, pl.ds(k * L, L)]
            s_idx = src_r[j, pl.ds(k * L, L)]
            vals = plsc.load_gather(dinv_v, [d_idx])
            plsc.addupdate_scatter(t1_v, [s_idx], vals)

    for b in range(NBUF):
        pltpu.async_copy(y_sh.at[src_r.at[b]], rows_v.at[b], gsem.at[b])

    def pipe_body(o, _):
        for b in range(NBUF):
            j = o * NBUF + b
            _t1_row(j)
            pltpu.make_async_copy(
                y_sh.at[src_r.at[j]], rows_v.at[b], gsem.at[b]).wait()
            pltpu.async_copy(rows_v.at[b], z_acc.at[dst_r.at[j]],
                             ssem.at[b], add=True)
            pltpu.make_async_copy(
                rows_v.at[b], z_acc.at[dst_r.at[j]], ssem.at[b]).wait()
            pltpu.async_copy(y_sh.at[src_r.at[j + NBUF]], rows_v.at[b],
                             gsem.at[b])
        return 0

    lax.fori_loop(0, OUT - 1, pipe_body, 0)
    for b in range(NBUF):
        j = (OUT - 1) * NBUF + b
        _t1_row(j)
        pltpu.make_async_copy(
            y_sh.at[src_r.at[j]], rows_v.at[b], gsem.at[b]).wait()
        pltpu.async_copy(rows_v.at[b], z_acc.at[dst_r.at[j]],
                         ssem.at[b], add=True)
        pltpu.make_async_copy(
            rows_v.at[b], z_acc.at[dst_r.at[j]], ssem.at[b]).wait()

    pltpu.sync_copy(t1_v, t1_out.at[wid])
    plsc.subcore_barrier()
    pltpu.sync_copy(z_acc.at[pl.ds(base, ZROWS)], sl16_v)
    pltpu.sync_copy(sl16_v, z_out.at[cid].at[pl.ds(base, ZROWS)])


@functools.partial(
    pl.kernel,
    out_type=(
        jax.ShapeDtypeStruct((NW, NPAD), jnp.float32),
        jax.ShapeDtypeStruct((NPAD,), jnp.float32),
    ),
    mesh=_mesh,
    compiler_params=_sc_params,
    scratch_types=[
        pltpu.VMEM((EPW_PAD,), jnp.int32),
        pltpu.VMEM((EPW_PAD,), jnp.int32),
        pltpu.VMEM((NPAD,), jnp.float32),
        pltpu.VMEM((NPAD,), jnp.float32),
        pltpu.VMEM((NW, ZROWS), jnp.float32),
        pltpu.VMEM((ZROWS,), jnp.float32),
        pltpu.VMEM((ZROWS,), jnp.float32),
        pltpu.VMEM_SHARED((NPAD,), jnp.float32),
        pltpu.SemaphoreType.DMA,
        pltpu.SemaphoreType.DMA,
        pltpu.SemaphoreType.DMA,
    ],
)
def _sc_t2(src_hbm, dst_hbm, t1p_hbm, dinv_hbm, t2_out, a_out,
           src_v, dst_v, g_v, acc_v, tmp_v, sacc_v, dv_v, g_sh,
           s_sem, d_sem, p_sem):
    cid = lax.axis_index("c")
    sid = lax.axis_index("s")
    wid = sid * NC + cid
    base = sid * ZROWS

    src_cp = pltpu.async_copy(src_hbm.at[wid], src_v, s_sem)
    dst_cp = pltpu.async_copy(dst_hbm.at[wid], dst_v, d_sem)
    t1_cp = pltpu.async_copy(t1p_hbm.at[:, pl.ds(base, ZROWS)], tmp_v,
                             p_sem)
    _zero_1d(acc_v, NPAD // L)
    t1_cp.wait()

    def rbody(i, _):
        sl = pl.ds(i * L, L)
        acc = tmp_v[0, sl]
        for p in range(1, NW):
            acc = acc + tmp_v[p, sl]
        sacc_v[sl] = acc
        return 0

    lax.fori_loop(0, ZROWS // L, rbody, 0)

    pltpu.sync_copy(dinv_hbm.at[pl.ds(base, ZROWS)], dv_v)
    for i in range(ZROWS // L):
        sl = pl.ds(i * L, L)
        dv = dv_v[sl]
        a = dv * (sacc_v[sl] + dv)
        sacc_v[sl] = a
        dv_v[sl] = dv * a
    pltpu.sync_copy(dv_v, g_sh.at[pl.ds(base, ZROWS)])

    @pl.when(cid == 0)
    def _():
        pltpu.sync_copy(sacc_v, a_out.at[pl.ds(base, ZROWS)])

    plsc.subcore_barrier()
    pltpu.sync_copy(g_sh, g_v)

    src_cp.wait()
    dst_cp.wait()

    def body(i, _):
        d_idx = dst_v[pl.ds(i * L, L)]
        s_idx = src_v[pl.ds(i * L, L)]
        vals = plsc.load_gather(g_v, [d_idx])
        plsc.addupdate_scatter(acc_v, [s_idx], vals)
        return 0

    lax.fori_loop(0, EPW_PAD // L, body, 0)
    pltpu.sync_copy(acc_v, t2_out.at[wid])


def _tc_final_body(xw_ref, dinv_ref, a_ref, z0_ref, z1_ref, t2T_ref, b1_ref,
                   w2_ref, w3_ref, wl_ref, b2_ref, b3_ref, bl_ref, out_ref):
    dinv = dinv_ref[...]
    a = a_ref[...]
    y = dinv * xw_ref[...]
    z = z0_ref[...] + z1_ref[...]
    h1 = jnp.maximum(dinv * (z + y) + b1_ref[...], 0.0)
    t2 = jnp.sum(t2T_ref[...], axis=1, keepdims=True)
    w = dinv * t2 + dinv * dinv * a
    u = jnp.sum(w * h1, axis=0, keepdims=True)
    s = jnp.sum(a, axis=0, keepdims=True)
    w3 = w3_ref[...]
    w23 = jnp.dot(w2_ref[...], w3, preferred_element_type=jnp.float32)
    pooled = (jnp.dot(u, w23, preferred_element_type=jnp.float32)
              + s * jnp.dot(b2_ref[...], w3,
                            preferred_element_type=jnp.float32)
              ) * (1.0 / N) + b3_ref[...]
    logit = jnp.dot(pooled, wl_ref[...],
                    preferred_element_type=jnp.float32) + bl_ref[...]
    out_ref[...] = jax.nn.sigmoid(logit)


_tc_final = pl.pallas_call(
    _tc_final_body,
    out_shape=jax.ShapeDtypeStruct((1, 1), jnp.float32),
)


def kernel(x, edge_index, batch, W1, b1, W2, b2, W3, b3, Wl, bl):
    del batch
    src = edge_index[0].astype(jnp.int32).reshape(NW, EPW)
    dst = edge_index[1].astype(jnp.int32).reshape(NW, EPW)
    pad = EPW_PAD - EPW
    src_p = jnp.pad(src, ((0, 0), (0, pad)))
    dst_p = jnp.pad(dst, ((0, 0), (0, pad)), constant_values=N)
    src3 = src_p.reshape(NW, RPW, 128)
    dst3 = dst_p.reshape(NW, RPW, 128)

    xw = _tc_xw(x, W1)
    deg_p = _sc_deg(dst_p)
    t1_p, z_p, dinv = _sc_edge(src3, dst3, deg_p, xw)
    t2_p, a = _sc_t2(src_p, dst_p, t1_p, dinv)
    out = _tc_final(xw, dinv.reshape(NPAD, 1), a.reshape(NPAD, 1),
                    z_p[0], z_p[1], t2_p.T, b1.reshape(1, H),
                    W2, W3, Wl, b2.reshape(1, H), b3.reshape(1, H),
                    bl.reshape(1, 1))
    return out
